# baseline (speedup 1.0000x reference)
import jax
import jax.numpy as jnp
from jax.experimental import pallas as pl
from jax.experimental.pallas import tpu as pltpu

HIDDEN = 1024
EXPERT_INTER = 1024
NUM_EXPERTS = 16
VOCAB = 100000
N_TOKENS = 128
SPLIT = 2
HALF = EXPERT_INTER // SPLIT


def _moe_body(tid_ref, x_ref, gatew_ref, upw_ref, dnw_ref, out_ref):
    e = pl.program_id(0)
    j = pl.program_id(1)

    @pl.when((e == 0) & (j == 0))
    def _init():
        out_ref[...] = jnp.zeros_like(out_ref)

    x = x_ref[...].astype(jnp.bfloat16)
    gate = jnp.dot(x, gatew_ref[0].astype(jnp.bfloat16),
                   preferred_element_type=jnp.float32)
    up = jnp.dot(x, upw_ref[0].astype(jnp.bfloat16),
                 preferred_element_type=jnp.float32)
    act = gate * jax.nn.sigmoid(gate) * up
    y = jnp.dot(act.astype(jnp.bfloat16), dnw_ref[0].astype(jnp.bfloat16),
                preferred_element_type=jnp.float32)

    tid = jnp.clip(tid_ref[...], 0, VOCAB - 1)
    eid = jax.lax.rem(tid, NUM_EXPERTS)
    mask = eid == e
    out_ref[...] += jnp.where(mask, y, 0.0)


def kernel(x, token_ids, gate_up_proj, down_proj):
    n = x.shape[0]
    tid2d = token_ids.reshape(n, 1).astype(jnp.int32)
    return pl.pallas_call(
        _moe_body,
        grid=(NUM_EXPERTS, SPLIT),
        in_specs=[
            pl.BlockSpec((n, 1), lambda e, j: (0, 0)),
            pl.BlockSpec((n, HIDDEN), lambda e, j: (0, 0)),
            pl.BlockSpec((1, HIDDEN, HALF), lambda e, j: (e, 0, j)),
            pl.BlockSpec((1, HIDDEN, HALF), lambda e, j: (e, 0, SPLIT + j)),
            pl.BlockSpec((1, HALF, HIDDEN), lambda e, j: (e, j, 0)),
        ],
        out_specs=pl.BlockSpec((n, HIDDEN), lambda e, j: (0, 0)),
        out_shape=jax.ShapeDtypeStruct((n, HIDDEN), jnp.float32),
        compiler_params=pltpu.CompilerParams(
            dimension_semantics=("arbitrary", "arbitrary"),
        ),
    )(tid2d, x, gate_up_proj, gate_up_proj, down_proj)


# speedup vs baseline: 1.0352x; 1.0352x over previous
import jax
import jax.numpy as jnp
from jax.experimental import pallas as pl
from jax.experimental.pallas import tpu as pltpu

HIDDEN = 1024
EXPERT_INTER = 1024
NUM_EXPERTS = 16
VOCAB = 100000
N_TOKENS = 128


def _moe_body(tid_ref, x_ref, gatew_ref, upw_ref, dnw_ref, out_ref):
    e = pl.program_id(0)

    @pl.when(e == 0)
    def _init():
        out_ref[...] = jnp.zeros_like(out_ref)

    x = x_ref[...].astype(jnp.bfloat16)
    gate = jnp.dot(x, gatew_ref[0].astype(jnp.bfloat16),
                   preferred_element_type=jnp.float32)
    up = jnp.dot(x, upw_ref[0].astype(jnp.bfloat16),
                 preferred_element_type=jnp.float32)
    act = gate * jax.nn.sigmoid(gate) * up
    y = jnp.dot(act.astype(jnp.bfloat16), dnw_ref[0].astype(jnp.bfloat16),
                preferred_element_type=jnp.float32)

    tid = jnp.clip(tid_ref[...], 0, VOCAB - 1)
    eid = jax.lax.rem(tid, NUM_EXPERTS)
    mask = eid == e
    out_ref[...] += jnp.where(mask, y, 0.0)


def kernel(x, token_ids, gate_up_proj, down_proj):
    n = x.shape[0]
    tid2d = token_ids.reshape(n, 1).astype(jnp.int32)
    return pl.pallas_call(
        _moe_body,
        grid=(NUM_EXPERTS,),
        in_specs=[
            pl.BlockSpec((n, 1), lambda e: (0, 0)),
            pl.BlockSpec((n, HIDDEN), lambda e: (0, 0)),
            pl.BlockSpec((1, HIDDEN, EXPERT_INTER), lambda e: (e, 0, 0)),
            pl.BlockSpec((1, HIDDEN, EXPERT_INTER), lambda e: (e, 0, 1)),
            pl.BlockSpec((1, EXPERT_INTER, HIDDEN), lambda e: (e, 0, 0)),
        ],
        out_specs=pl.BlockSpec((n, HIDDEN), lambda e: (0, 0)),
        out_shape=jax.ShapeDtypeStruct((n, HIDDEN), jnp.float32),
        compiler_params=pltpu.CompilerParams(
            dimension_semantics=("arbitrary",),
        ),
    )(tid2d, x, gate_up_proj, gate_up_proj, down_proj)
